# baseline (speedup 1.0000x reference)
import jax
import jax.numpy as jnp
from jax import lax
from jax.experimental import pallas as pl
from jax.experimental.pallas import tpu as pltpu
from jax.experimental.pallas import tpu_sc as plsc

D = 64
B = 16384
NC = 2
NS = 16
NW = NC * NS
BPW = B // NW
L = 16


def _body(users_hbm, pos_hbm, neg_hbm, utab_hbm, itab_hbm,
          pos_out, neg_out,
          uidx, pidx, nidx, urows, prows, nrows, psc, nsc, tp, tn,
          sem_u, sem_p, sem_n):
    wid = lax.axis_index("s") * NC + lax.axis_index("c")
    base = wid * BPW

    pltpu.sync_copy(users_hbm.at[pl.ds(base, BPW)], uidx)
    pltpu.sync_copy(pos_hbm.at[pl.ds(base, BPW)], pidx)
    pltpu.sync_copy(neg_hbm.at[pl.ds(base, BPW)], nidx)

    cu = pltpu.async_copy(utab_hbm.at[uidx], urows, sem_u)
    cp = pltpu.async_copy(itab_hbm.at[pidx], prows, sem_p)
    cn = pltpu.async_copy(itab_hbm.at[nidx], nrows, sem_n)
    cu.wait()
    cp.wait()
    cn.wait()

    tcol = lax.iota(jnp.int32, L) * (L + 1)

    def block(j, carry):
        i0 = j * L
        for k in range(L):
            i = i0 + k
            accp = jnp.zeros((L,), jnp.float32)
            accn = jnp.zeros((L,), jnp.float32)
            for q in range(D // L):
                u = urows[i, pl.ds(q * L, L)]
                accp = accp + u * prows[i, pl.ds(q * L, L)]
                accn = accn + u * nrows[i, pl.ds(q * L, L)]
            plsc.store_scatter(tp, [tcol + k], accp)
            plsc.store_scatter(tn, [tcol + k], accn)
        sp = jnp.zeros((L,), jnp.float32)
        sn = jnp.zeros((L,), jnp.float32)
        for l in range(L):
            sp = sp + tp[pl.ds(l * (L + 1), L)]
            sn = sn + tn[pl.ds(l * (L + 1), L)]
        psc[pl.ds(i0, L)] = sp
        nsc[pl.ds(i0, L)] = sn
        return carry

    lax.fori_loop(0, BPW // L, block, 0, unroll=False)

    pltpu.sync_copy(psc, pos_out.at[pl.ds(base, BPW)])
    pltpu.sync_copy(nsc, neg_out.at[pl.ds(base, BPW)])


@jax.jit
def kernel(users, pos_items, neg_items, user_table, item_table):
    mesh = plsc.VectorSubcoreMesh(core_axis_name="c", subcore_axis_name="s",
                                  num_cores=NC, num_subcores=NS)
    k = pl.kernel(
        _body,
        out_type=(jax.ShapeDtypeStruct((B,), jnp.float32),
                  jax.ShapeDtypeStruct((B,), jnp.float32)),
        mesh=mesh,
        scratch_types=[
            pltpu.VMEM((BPW,), jnp.int32),
            pltpu.VMEM((BPW,), jnp.int32),
            pltpu.VMEM((BPW,), jnp.int32),
            pltpu.VMEM((BPW, D), jnp.float32),
            pltpu.VMEM((BPW, D), jnp.float32),
            pltpu.VMEM((BPW, D), jnp.float32),
            pltpu.VMEM((BPW,), jnp.float32),
            pltpu.VMEM((BPW,), jnp.float32),
            pltpu.VMEM((L * (L + 1),), jnp.float32),
            pltpu.VMEM((L * (L + 1),), jnp.float32),
            pltpu.SemaphoreType.DMA,
            pltpu.SemaphoreType.DMA,
            pltpu.SemaphoreType.DMA,
        ],
        compiler_params=pltpu.CompilerParams(needs_layout_passes=False,
                                             use_tc_tiling_on_sc=False),
        name="bprmf_sc_score",
    )
    return k(users, pos_items, neg_items, user_table, item_table)
